# baseline (speedup 1.0000x reference)
import jax
import jax.numpy as jnp
from jax import lax
from jax.experimental import pallas as pl
from jax.experimental.pallas import tpu as pltpu
from jax.experimental.pallas import tpu_sc as plsc

N = 10000
NE = 320000
D = 128
DE = 16
PK = 32

NC = 2
NS = 16
NW = NC * NS
CH = 80
CPT = NE // (CH * NW)
NPAIR = CPT // 2
WB = 624
TAIL = N - NS * WB
TAIL0 = NS * WB

_DN = lax.GatherDimensionNumbers(
    offset_dims=(), collapsed_slice_dims=(0,), start_index_map=(0,))


def _lane_bcast(wv, j):
    return lax.gather(wv, jnp.full((16, 1), j, jnp.int32), _DN, (1,),
                      mode=lax.GatherScatterMode.PROMISE_IN_BOUNDS)


def _pool_body(v_ref, aw_ref, ab_ref, out_ref):
    x = v_ref[...]
    x = jnp.where(x >= 0.0, x, 0.2 * x)
    p = lax.dot_general(x, aw_ref[...], (((1,), (1,)), ((), ())),
                        preferred_element_type=jnp.float32)
    out_ref[...] = jnp.maximum(p + ab_ref[...], 0.0)


def _pool(V, A_w, A_b):
    return pl.pallas_call(
        _pool_body,
        out_shape=jax.ShapeDtypeStruct((N, D), jnp.float32),
    )(V, A_w, A_b.reshape(1, D))


def _zero_init(z_hbm, acc_sh, sid):
    row0 = sid * WB
    pltpu.sync_copy(z_hbm.at[pl.ds(row0, WB)], acc_sh.at[pl.ds(row0, WB)])

    @pl.when(sid == NS - 1)
    def _zt():
        pltpu.sync_copy(z_hbm.at[pl.ds(TAIL0, TAIL)],
                        acc_sh.at[pl.ds(TAIL0, TAIL)])


def _writeback(acc_sh, out_hbm, cid, sid):
    row0 = sid * WB
    pltpu.sync_copy(acc_sh.at[pl.ds(row0, WB)],
                    out_hbm.at[cid, pl.ds(row0, WB)])

    @pl.when(sid == NS - 1)
    def _wt():
        pltpu.sync_copy(acc_sh.at[pl.ds(TAIL0, TAIL)],
                        out_hbm.at[cid, pl.ds(TAIL0, TAIL)])


def _s1_body(pooled_hbm, src_hbm, dst_hbm, w_hbm, z1_hbm, s1_out,
             s1_sh, srcv0, srcv1, dstv0, dstv1, wvm0, wvm1, rows0, rows1,
             sa0, sa1, sb0, sb1, sd0, sd1):
    cid = lax.axis_index("c")
    sid = lax.axis_index("s")
    wid = cid * NS + sid
    base0 = wid * CPT * CH

    _zero_init(z1_hbm, s1_sh, sid)
    plsc.subcore_barrier()

    def issue_a(slot, ci):
        srcv, sem = (srcv0, sa0) if slot == 0 else (srcv1, sa1)
        b = base0 + ci * CH
        pltpu.async_copy(src_hbm.at[pl.ds(b, CH)], srcv, sem)

    def wait_a(slot):
        srcv, sem = (srcv0, sa0) if slot == 0 else (srcv1, sa1)
        pltpu.make_async_copy(src_hbm.at[pl.ds(0, CH)], srcv, sem).wait()

    def issue_b(slot, ci):
        srcv, dstv, wvm, rows, sem = (
            (srcv0, dstv0, wvm0, rows0, sb0) if slot == 0
            else (srcv1, dstv1, wvm1, rows1, sb1))
        b = base0 + ci * CH
        pltpu.async_copy(pooled_hbm.at[srcv], rows, sem)
        pltpu.async_copy(dst_hbm.at[pl.ds(b, CH)], dstv, sem)
        pltpu.async_copy(w_hbm.at[pl.ds(b, CH)], wvm, sem)

    def wait_b(slot):
        srcv, dstv, wvm, rows, sem = (
            (srcv0, dstv0, wvm0, rows0, sb0) if slot == 0
            else (srcv1, dstv1, wvm1, rows1, sb1))
        pltpu.make_async_copy(pooled_hbm.at[srcv], rows, sem).wait()
        pltpu.make_async_copy(dst_hbm.at[pl.ds(0, CH)], dstv, sem).wait()
        pltpu.make_async_copy(w_hbm.at[pl.ds(0, CH)], wvm, sem).wait()

    def compute(slot):
        wvm, rows = (wvm0, rows0) if slot == 0 else (wvm1, rows1)

        def _group(g, _):
            wv = wvm[pl.ds(g * 16, 16)]
            for h in range(4):
                ws = [_lane_bcast(wv, h * 4 + t) for t in range(4)]
                es = [g * 16 + h * 4 + t for t in range(4)]
                vals = [[rows[es[t], pl.ds(k * 16, 16)]
                         for k in range(D // 16)] for t in range(4)]
                for t in range(4):
                    for k in range(D // 16):
                        rows[es[t], pl.ds(k * 16, 16)] = vals[t][k] * ws[t]
            return 0

        lax.fori_loop(0, CH // 16, _group, 0)

    def issue_d(slot):
        dstv, rows, sem = (dstv0, rows0, sd0) if slot == 0 else (dstv1,
                                                                 rows1, sd1)
        pltpu.async_copy(rows, s1_sh.at[dstv], sem, add=True)

    def wait_d(slot):
        dstv, rows, sem = (dstv0, rows0, sd0) if slot == 0 else (dstv1,
                                                                 rows1, sd1)
        pltpu.make_async_copy(rows, s1_sh.at[dstv], sem).wait()

    issue_a(0, 0)
    issue_a(1, 1)
    wait_a(0)
    issue_b(0, 0)

    def _pair(p, _):
        wait_b(0)
        wait_a(1)

        @pl.when(p > 0)
        def _():
            wait_d(1)

        issue_b(1, 2 * p + 1)
        issue_a(0, 2 * p + 2)
        compute(0)
        issue_d(0)
        wait_b(1)
        wait_a(0)
        wait_d(0)
        issue_b(0, 2 * p + 2)

        @pl.when(p < NPAIR - 1)
        def _():
            issue_a(1, 2 * p + 3)

        compute(1)
        issue_d(1)
        return 0

    lax.fori_loop(0, NPAIR, _pair, 0)

    wait_b(0)
    compute(0)
    issue_d(0)
    wait_d(0)
    wait_d(1)

    plsc.subcore_barrier()
    _writeback(s1_sh, s1_out, cid, sid)


def _sc_s1(pooled, src, dst, w):
    mesh = plsc.VectorSubcoreMesh(core_axis_name="c", subcore_axis_name="s")
    f = pl.kernel(
        _s1_body,
        out_type=jax.ShapeDtypeStruct((NC, N, D), jnp.float32),
        mesh=mesh,
        compiler_params=pltpu.CompilerParams(use_tc_tiling_on_sc=False),
        scratch_types=[
            pltpu.VMEM_SHARED((N, D), jnp.float32),
            pltpu.VMEM((CH,), jnp.int32),
            pltpu.VMEM((CH,), jnp.int32),
            pltpu.VMEM((CH,), jnp.int32),
            pltpu.VMEM((CH,), jnp.int32),
            pltpu.VMEM((CH,), jnp.float32),
            pltpu.VMEM((CH,), jnp.float32),
            pltpu.VMEM((CH, D), jnp.float32),
            pltpu.VMEM((CH, D), jnp.float32),
            pltpu.SemaphoreType.DMA,
            pltpu.SemaphoreType.DMA,
            pltpu.SemaphoreType.DMA,
            pltpu.SemaphoreType.DMA,
            pltpu.SemaphoreType.DMA,
            pltpu.SemaphoreType.DMA,
        ],
    )
    z1 = jnp.zeros((N, D), jnp.float32)
    return f(pooled, src, dst, w, z1)


CH2 = 400
CPT2 = NE // (CH2 * NW)
NPAIR2 = CPT2 // 2
SPLITS = ((0, 96), (96, 96), (192, 96), (288, 112))


def _s2x_body(dst_hbm, w_hbm, e_hbm, z2_hbm, s2x_out,
              s2x_sh, da0, db0, dc0, dd0, da1, db1, dc1, dd1,
              wvm0, wvm1, ev0, ev1, pk0, pk1,
              sa0, sa1, sb0, sb1, sd0, sd1):
    cid = lax.axis_index("c")
    sid = lax.axis_index("s")
    wid = cid * NS + sid
    base0 = wid * CPT2 * CH2
    dsts0 = (da0, db0, dc0, dd0)
    dsts1 = (da1, db1, dc1, dd1)

    _zero_init(z2_hbm, s2x_sh, sid)
    plsc.subcore_barrier()

    lane = lax.iota(jnp.int32, 16)

    def issue_a(slot, ci):
        ev, wvm, sem = (ev0, wvm0, sa0) if slot == 0 else (ev1, wvm1, sa1)
        b = base0 + ci * CH2
        pltpu.async_copy(e_hbm.at[pl.ds(b, CH2)], ev, sem)
        pltpu.async_copy(w_hbm.at[pl.ds(b, CH2)], wvm, sem)

    def wait_a(slot):
        ev, wvm, sem = (ev0, wvm0, sa0) if slot == 0 else (ev1, wvm1, sa1)
        pltpu.make_async_copy(e_hbm.at[pl.ds(0, CH2)], ev, sem).wait()
        pltpu.make_async_copy(w_hbm.at[pl.ds(0, CH2)], wvm, sem).wait()

    def issue_b(slot, ci):
        dsts, sem = (dsts0, sb0) if slot == 0 else (dsts1, sb1)
        b = base0 + ci * CH2
        for (off, ln), dv in zip(SPLITS, dsts):
            pltpu.async_copy(dst_hbm.at[pl.ds(b + off, ln)], dv, sem)

    def wait_b(slot):
        dsts, sem = (dsts0, sb0) if slot == 0 else (dsts1, sb1)
        for (off, ln), dv in zip(SPLITS, dsts):
            pltpu.make_async_copy(dst_hbm.at[pl.ds(0, ln)], dv, sem).wait()

    def compute(slot):
        wvm, ev, pk = (wvm0, ev0, pk0) if slot == 0 else (wvm1, ev1, pk1)

        def _group(g, _):
            wv = wvm[pl.ds(g * 16, 16)]
            for h in range(4):
                ws = [_lane_bcast(wv, h * 4 + t) for t in range(4)]
                es = [g * 16 + h * 4 + t for t in range(4)]
                evs = [ev[es[t], :] for t in range(4)]
                for t in range(4):
                    pk[es[t], pl.ds(0, 16)] = evs[t] * ws[t]
                    pk[es[t], pl.ds(16, 16)] = jnp.where(
                        lane == 0, ws[t], jnp.where(lane == 1, 1.0, 0.0))
            return 0

        lax.fori_loop(0, CH2 // 16, _group, 0)

    def issue_d(slot):
        dsts, pk, sem = ((dsts0, pk0, sd0) if slot == 0
                         else (dsts1, pk1, sd1))
        for (off, ln), dv in zip(SPLITS, dsts):
            pltpu.async_copy(pk.at[pl.ds(off, ln)], s2x_sh.at[dv], sem,
                             add=True)

    def wait_d(slot):
        dsts, pk, sem = ((dsts0, pk0, sd0) if slot == 0
                         else (dsts1, pk1, sd1))
        for (off, ln), dv in zip(SPLITS, dsts):
            pltpu.make_async_copy(pk.at[pl.ds(off, ln)], s2x_sh.at[dv],
                                  sem).wait()

    issue_a(0, 0)
    issue_a(1, 1)
    wait_a(0)
    issue_b(0, 0)

    def _pair(p, _):
        wait_b(0)
        compute(0)
        issue_d(0)
        wait_a(1)

        @pl.when(p > 0)
        def _():
            wait_d(1)

        issue_b(1, 2 * p + 1)
        issue_a(0, 2 * p + 2)
        wait_b(1)
        compute(1)
        issue_d(1)
        wait_a(0)
        wait_d(0)
        issue_b(0, 2 * p + 2)

        @pl.when(p < NPAIR2 - 1)
        def _():
            issue_a(1, 2 * p + 3)

        return 0

    lax.fori_loop(0, NPAIR2, _pair, 0)

    wait_b(0)
    compute(0)
    issue_d(0)
    wait_d(0)
    wait_d(1)

    plsc.subcore_barrier()
    _writeback(s2x_sh, s2x_out, cid, sid)


def _sc_s2x(dst, w, E):
    mesh = plsc.VectorSubcoreMesh(core_axis_name="c", subcore_axis_name="s")
    f = pl.kernel(
        _s2x_body,
        out_type=jax.ShapeDtypeStruct((NC, N, PK), jnp.float32),
        mesh=mesh,
        compiler_params=pltpu.CompilerParams(use_tc_tiling_on_sc=False),
        scratch_types=(
            [pltpu.VMEM_SHARED((N, PK), jnp.float32)]
            + [pltpu.VMEM((ln,), jnp.int32)
               for _s in range(2) for (_o, ln) in SPLITS]
            + [
                pltpu.VMEM((CH2,), jnp.float32),
                pltpu.VMEM((CH2,), jnp.float32),
                pltpu.VMEM((CH2, DE), jnp.float32),
                pltpu.VMEM((CH2, DE), jnp.float32),
                pltpu.VMEM((CH2, PK), jnp.float32),
                pltpu.VMEM((CH2, PK), jnp.float32),
            ]
            + [pltpu.SemaphoreType.DMA] * 6
        ),
    )
    z2 = jnp.zeros((N, PK), jnp.float32)
    return f(dst, w, E, z2)


def _final_body(s1_ref, s2x_ref, vin_ref, mw_ref, mb_ref, w1_ref, w2_ref,
                wb_ref, out_ref):
    s1 = s1_ref[0] + s1_ref[1]
    s2x = s2x_ref[0] + s2x_ref[1]
    s2 = s2x[:, :DE]
    s3 = s2x[:, DE:DE + 1]
    cnt = s2x[:, DE + 1:DE + 2]
    proj = lax.dot_general(s2, mw_ref[...], (((1,), (1,)), ((), ())),
                           preferred_element_type=jnp.float32)
    summed = s1 + proj + s3 * mb_ref[...]
    agg = summed / jnp.maximum(cnt, 1.0)
    o = lax.dot_general(agg, w1_ref[...], (((1,), (1,)), ((), ())),
                        preferred_element_type=jnp.float32)
    o = o + lax.dot_general(vin_ref[...], w2_ref[...],
                            (((1,), (1,)), ((), ())),
                            preferred_element_type=jnp.float32)
    out_ref[...] = jnp.maximum(o + wb_ref[...], 0.0)


def _final(S1p, S2xp, V_in, M_w, M_b, W1, W2, W_b):
    return pl.pallas_call(
        _final_body,
        out_shape=jax.ShapeDtypeStruct((N, D), jnp.float32),
    )(S1p, S2xp, V_in, M_w, M_b.reshape(1, D), W1, W2, W_b.reshape(1, D))


def kernel(V, V_in, E, edge_attr, edge_index, A_w, A_b, M_w, M_b, W_w, W_b):
    w = edge_attr[:, 0]
    src = edge_index[0]
    dst = edge_index[1]
    pooled = _pool(V, A_w, A_b)
    S1p = _sc_s1(pooled, src, dst, w)
    S2xp = _sc_s2x(dst, w, E)
    return _final(S1p, S2xp, V_in, M_w, M_b, W_w[:, :D], W_w[:, D:], W_b)


# speedup vs baseline: 1.1048x; 1.1048x over previous
import jax
import jax.numpy as jnp
from jax import lax
from jax.experimental import pallas as pl
from jax.experimental.pallas import tpu as pltpu
from jax.experimental.pallas import tpu_sc as plsc

N = 10000
NE = 320000
D = 128
DE = 16
PK = 32

NC = 2
NS = 16
NW = NC * NS
CH = 80
CPT = NE // (CH * NW)
NPAIR = CPT // 2
WB = 624
TAIL = N - NS * WB
TAIL0 = NS * WB

_DN = lax.GatherDimensionNumbers(
    offset_dims=(), collapsed_slice_dims=(0,), start_index_map=(0,))


def _lane_bcast(wv, j):
    return lax.gather(wv, jnp.full((16, 1), j, jnp.int32), _DN, (1,),
                      mode=lax.GatherScatterMode.PROMISE_IN_BOUNDS)


def _pool_body(v_ref, aw_ref, ab_ref, out_ref):
    x = v_ref[...]
    x = jnp.where(x >= 0.0, x, 0.2 * x)
    p = lax.dot_general(x, aw_ref[...], (((1,), (1,)), ((), ())),
                        preferred_element_type=jnp.float32)
    p = jnp.maximum(p + ab_ref[...], 0.0)
    lo = lax.bitcast_convert_type(p[:, :D // 2].astype(jnp.bfloat16),
                                  jnp.uint16).astype(jnp.uint32)
    hi = lax.bitcast_convert_type(p[:, D // 2:].astype(jnp.bfloat16),
                                  jnp.uint16).astype(jnp.uint32)
    out_ref[...] = lax.bitcast_convert_type(lo | (hi << 16), jnp.int32)


def _pool(V, A_w, A_b):
    return pl.pallas_call(
        _pool_body,
        out_shape=jax.ShapeDtypeStruct((N, D // 2), jnp.int32),
    )(V, A_w, A_b.reshape(1, D))


def _zero_init(z_hbm, acc_sh, sid):
    row0 = sid * WB
    pltpu.sync_copy(z_hbm.at[pl.ds(row0, WB)], acc_sh.at[pl.ds(row0, WB)])

    @pl.when(sid == NS - 1)
    def _zt():
        pltpu.sync_copy(z_hbm.at[pl.ds(TAIL0, TAIL)],
                        acc_sh.at[pl.ds(TAIL0, TAIL)])


def _writeback(acc_sh, out_hbm, cid, sid):
    row0 = sid * WB
    pltpu.sync_copy(acc_sh.at[pl.ds(row0, WB)],
                    out_hbm.at[cid, pl.ds(row0, WB)])

    @pl.when(sid == NS - 1)
    def _wt():
        pltpu.sync_copy(acc_sh.at[pl.ds(TAIL0, TAIL)],
                        out_hbm.at[cid, pl.ds(TAIL0, TAIL)])


def _s1_body(pooled_hbm, src_hbm, dst_hbm, w_hbm, z1_hbm, s1_out,
             s1_sh, srcv0, srcv1, dstv0, dstv1, wvm0, wvm1, rbf0, rbf1,
             r32_0, r32_1, sa0, sa1, sb0, sb1, sd0, sd1):
    cid = lax.axis_index("c")
    sid = lax.axis_index("s")
    wid = cid * NS + sid
    base0 = wid * CPT * CH

    _zero_init(z1_hbm, s1_sh, sid)
    plsc.subcore_barrier()

    def issue_a(slot, ci):
        srcv, sem = (srcv0, sa0) if slot == 0 else (srcv1, sa1)
        b = base0 + ci * CH
        pltpu.async_copy(src_hbm.at[pl.ds(b, CH)], srcv, sem)

    def wait_a(slot):
        srcv, sem = (srcv0, sa0) if slot == 0 else (srcv1, sa1)
        pltpu.make_async_copy(src_hbm.at[pl.ds(0, CH)], srcv, sem).wait()

    def issue_b(slot, ci):
        srcv, dstv, wvm, rbf, sem = (
            (srcv0, dstv0, wvm0, rbf0, sb0) if slot == 0
            else (srcv1, dstv1, wvm1, rbf1, sb1))
        b = base0 + ci * CH
        pltpu.async_copy(pooled_hbm.at[srcv], rbf, sem)
        pltpu.async_copy(dst_hbm.at[pl.ds(b, CH)], dstv, sem)
        pltpu.async_copy(w_hbm.at[pl.ds(b, CH)], wvm, sem)

    def wait_b(slot):
        srcv, dstv, wvm, rbf, sem = (
            (srcv0, dstv0, wvm0, rbf0, sb0) if slot == 0
            else (srcv1, dstv1, wvm1, rbf1, sb1))
        pltpu.make_async_copy(pooled_hbm.at[srcv], rbf, sem).wait()
        pltpu.make_async_copy(dst_hbm.at[pl.ds(0, CH)], dstv, sem).wait()
        pltpu.make_async_copy(w_hbm.at[pl.ds(0, CH)], wvm, sem).wait()

    def compute(slot):
        wvm, rbf, r32 = ((wvm0, rbf0, r32_0) if slot == 0
                         else (wvm1, rbf1, r32_1))

        def _group(g, _):
            wv = wvm[pl.ds(g * 16, 16)]
            for h in range(4):
                ws = [_lane_bcast(wv, h * 4 + t) for t in range(4)]
                es = [g * 16 + h * 4 + t for t in range(4)]
                pks = [[rbf[es[t], pl.ds(m * 16, 16)]
                        for m in range(D // 32)] for t in range(4)]
                himask = jnp.full((16,), -65536, jnp.int32)
                for t in range(4):
                    for m in range(D // 32):
                        x = pks[t][m]
                        a = plsc.bitcast(x << 16, jnp.float32)
                        bb = plsc.bitcast(x & himask, jnp.float32)
                        r32[es[t], pl.ds(m * 16, 16)] = a * ws[t]
                        r32[es[t], pl.ds(D // 2 + m * 16, 16)] = bb * ws[t]
            return 0

        lax.fori_loop(0, CH // 16, _group, 0)

    def issue_d(slot):
        dstv, r32, sem = (dstv0, r32_0, sd0) if slot == 0 else (dstv1,
                                                                r32_1, sd1)
        pltpu.async_copy(r32, s1_sh.at[dstv], sem, add=True)

    def wait_d(slot):
        dstv, r32, sem = (dstv0, r32_0, sd0) if slot == 0 else (dstv1,
                                                                r32_1, sd1)
        pltpu.make_async_copy(r32, s1_sh.at[dstv], sem).wait()

    issue_a(0, 0)
    issue_a(1, 1)
    wait_a(0)
    issue_b(0, 0)

    def _pair(p, _):
        wait_b(0)
        wait_a(1)
        issue_b(1, 2 * p + 1)
        issue_a(0, 2 * p + 2)

        @pl.when(p > 0)
        def _():
            wait_d(0)

        compute(0)
        issue_d(0)
        wait_b(1)
        wait_a(0)
        issue_b(0, 2 * p + 2)

        @pl.when(p < NPAIR - 1)
        def _():
            issue_a(1, 2 * p + 3)

        @pl.when(p > 0)
        def _():
            wait_d(1)

        compute(1)
        issue_d(1)
        return 0

    lax.fori_loop(0, NPAIR, _pair, 0)

    wait_b(0)
    wait_d(0)
    compute(0)
    issue_d(0)
    wait_d(0)
    wait_d(1)

    plsc.subcore_barrier()
    _writeback(s1_sh, s1_out, cid, sid)


def _sc_s1(pooled, src, dst, w):
    mesh = plsc.VectorSubcoreMesh(core_axis_name="c", subcore_axis_name="s")
    f = pl.kernel(
        _s1_body,
        out_type=jax.ShapeDtypeStruct((NC, N, D), jnp.float32),
        mesh=mesh,
        compiler_params=pltpu.CompilerParams(use_tc_tiling_on_sc=False,
                                             needs_layout_passes=False),
        scratch_types=[
            pltpu.VMEM_SHARED((N, D), jnp.float32),
            pltpu.VMEM((CH,), jnp.int32),
            pltpu.VMEM((CH,), jnp.int32),
            pltpu.VMEM((CH,), jnp.int32),
            pltpu.VMEM((CH,), jnp.int32),
            pltpu.VMEM((CH,), jnp.float32),
            pltpu.VMEM((CH,), jnp.float32),
            pltpu.VMEM((CH, D // 2), jnp.int32),
            pltpu.VMEM((CH, D // 2), jnp.int32),
            pltpu.VMEM((CH, D), jnp.float32),
            pltpu.VMEM((CH, D), jnp.float32),
            pltpu.SemaphoreType.DMA,
            pltpu.SemaphoreType.DMA,
            pltpu.SemaphoreType.DMA,
            pltpu.SemaphoreType.DMA,
            pltpu.SemaphoreType.DMA,
            pltpu.SemaphoreType.DMA,
        ],
    )
    z1 = jnp.zeros((N, D), jnp.float32)
    return f(pooled, src, dst, w, z1)


CH2 = 400
CPT2 = NE // (CH2 * NW)
NPAIR2 = CPT2 // 2
SPLITS = ((0, 96), (96, 96), (192, 96), (288, 112))


def _s2x_body(dst_hbm, w_hbm, e_hbm, z2_hbm, s2x_out,
              s2x_sh, da0, db0, dc0, dd0, da1, db1, dc1, dd1,
              wvm0, wvm1, ev0, ev1, pk0, pk1,
              sa0, sa1, sb0, sb1, sd0, sd1):
    cid = lax.axis_index("c")
    sid = lax.axis_index("s")
    wid = cid * NS + sid
    base0 = wid * CPT2 * CH2
    dsts0 = (da0, db0, dc0, dd0)
    dsts1 = (da1, db1, dc1, dd1)

    _zero_init(z2_hbm, s2x_sh, sid)
    plsc.subcore_barrier()

    lane = lax.iota(jnp.int32, 16)

    def issue_a(slot, ci):
        ev, wvm, sem = (ev0, wvm0, sa0) if slot == 0 else (ev1, wvm1, sa1)
        b = base0 + ci * CH2
        pltpu.async_copy(e_hbm.at[pl.ds(b, CH2)], ev, sem)
        pltpu.async_copy(w_hbm.at[pl.ds(b, CH2)], wvm, sem)

    def wait_a(slot):
        ev, wvm, sem = (ev0, wvm0, sa0) if slot == 0 else (ev1, wvm1, sa1)
        pltpu.make_async_copy(e_hbm.at[pl.ds(0, CH2)], ev, sem).wait()
        pltpu.make_async_copy(w_hbm.at[pl.ds(0, CH2)], wvm, sem).wait()

    def issue_b(slot, ci):
        dsts, sem = (dsts0, sb0) if slot == 0 else (dsts1, sb1)
        b = base0 + ci * CH2
        for (off, ln), dv in zip(SPLITS, dsts):
            pltpu.async_copy(dst_hbm.at[pl.ds(b + off, ln)], dv, sem)

    def wait_b(slot):
        dsts, sem = (dsts0, sb0) if slot == 0 else (dsts1, sb1)
        for (off, ln), dv in zip(SPLITS, dsts):
            pltpu.make_async_copy(dst_hbm.at[pl.ds(0, ln)], dv, sem).wait()

    def compute(slot):
        wvm, ev, pk = (wvm0, ev0, pk0) if slot == 0 else (wvm1, ev1, pk1)

        def _group(g, _):
            wv = wvm[pl.ds(g * 16, 16)]
            for h in range(4):
                ws = [_lane_bcast(wv, h * 4 + t) for t in range(4)]
                es = [g * 16 + h * 4 + t for t in range(4)]
                evs = [ev[es[t], :] for t in range(4)]
                for t in range(4):
                    pk[es[t], pl.ds(0, 16)] = evs[t] * ws[t]
                    pk[es[t], pl.ds(16, 16)] = jnp.where(
                        lane == 0, ws[t], jnp.where(lane == 1, 1.0, 0.0))
            return 0

        lax.fori_loop(0, CH2 // 16, _group, 0)

    def issue_d(slot):
        dsts, pk, sem = ((dsts0, pk0, sd0) if slot == 0
                         else (dsts1, pk1, sd1))
        for (off, ln), dv in zip(SPLITS, dsts):
            pltpu.async_copy(pk.at[pl.ds(off, ln)], s2x_sh.at[dv], sem,
                             add=True)

    def wait_d(slot):
        dsts, pk, sem = ((dsts0, pk0, sd0) if slot == 0
                         else (dsts1, pk1, sd1))
        for (off, ln), dv in zip(SPLITS, dsts):
            pltpu.make_async_copy(pk.at[pl.ds(off, ln)], s2x_sh.at[dv],
                                  sem).wait()

    issue_a(0, 0)
    issue_a(1, 1)
    wait_a(0)
    issue_b(0, 0)

    def _pair(p, _):
        wait_b(0)
        compute(0)
        issue_d(0)
        wait_a(1)

        @pl.when(p > 0)
        def _():
            wait_d(1)

        issue_b(1, 2 * p + 1)
        issue_a(0, 2 * p + 2)
        wait_b(1)
        compute(1)
        issue_d(1)
        wait_a(0)
        wait_d(0)
        issue_b(0, 2 * p + 2)

        @pl.when(p < NPAIR2 - 1)
        def _():
            issue_a(1, 2 * p + 3)

        return 0

    lax.fori_loop(0, NPAIR2, _pair, 0)

    wait_b(0)
    compute(0)
    issue_d(0)
    wait_d(0)
    wait_d(1)

    plsc.subcore_barrier()
    _writeback(s2x_sh, s2x_out, cid, sid)


def _sc_s2x(dst, w, E):
    mesh = plsc.VectorSubcoreMesh(core_axis_name="c", subcore_axis_name="s")
    f = pl.kernel(
        _s2x_body,
        out_type=jax.ShapeDtypeStruct((NC, N, PK), jnp.float32),
        mesh=mesh,
        compiler_params=pltpu.CompilerParams(use_tc_tiling_on_sc=False),
        scratch_types=(
            [pltpu.VMEM_SHARED((N, PK), jnp.float32)]
            + [pltpu.VMEM((ln,), jnp.int32)
               for _s in range(2) for (_o, ln) in SPLITS]
            + [
                pltpu.VMEM((CH2,), jnp.float32),
                pltpu.VMEM((CH2,), jnp.float32),
                pltpu.VMEM((CH2, DE), jnp.float32),
                pltpu.VMEM((CH2, DE), jnp.float32),
                pltpu.VMEM((CH2, PK), jnp.float32),
                pltpu.VMEM((CH2, PK), jnp.float32),
            ]
            + [pltpu.SemaphoreType.DMA] * 6
        ),
    )
    z2 = jnp.zeros((N, PK), jnp.float32)
    return f(dst, w, E, z2)


def _final_body(s1_ref, s2x_ref, vin_ref, mw_ref, mb_ref, w1_ref, w2_ref,
                wb_ref, out_ref):
    s1 = s1_ref[0] + s1_ref[1]
    s2x = s2x_ref[0] + s2x_ref[1]
    s2 = s2x[:, :DE]
    s3 = s2x[:, DE:DE + 1]
    cnt = s2x[:, DE + 1:DE + 2]
    proj = lax.dot_general(s2, mw_ref[...], (((1,), (1,)), ((), ())),
                           preferred_element_type=jnp.float32)
    summed = s1 + proj + s3 * mb_ref[...]
    agg = summed / jnp.maximum(cnt, 1.0)
    o = lax.dot_general(agg, w1_ref[...], (((1,), (1,)), ((), ())),
                        preferred_element_type=jnp.float32)
    o = o + lax.dot_general(vin_ref[...], w2_ref[...],
                            (((1,), (1,)), ((), ())),
                            preferred_element_type=jnp.float32)
    out_ref[...] = jnp.maximum(o + wb_ref[...], 0.0)


def _final(S1p, S2xp, V_in, M_w, M_b, W1, W2, W_b):
    return pl.pallas_call(
        _final_body,
        out_shape=jax.ShapeDtypeStruct((N, D), jnp.float32),
    )(S1p, S2xp, V_in, M_w, M_b.reshape(1, D), W1, W2, W_b.reshape(1, D))


def kernel(V, V_in, E, edge_attr, edge_index, A_w, A_b, M_w, M_b, W_w, W_b):
    w = edge_attr[:, 0]
    src = edge_index[0]
    dst = edge_index[1]
    pooled = _pool(V, A_w, A_b)
    S1p = _sc_s1(pooled, src, dst, w)
    S2xp = _sc_s2x(dst, w, E)
    return _final(S1p, S2xp, V_in, M_w, M_b, W_w[:, :D], W_w[:, D:], W_b)


# speedup vs baseline: 1.1055x; 1.0007x over previous
import jax
import jax.numpy as jnp
from jax import lax
from jax.experimental import pallas as pl
from jax.experimental.pallas import tpu as pltpu
from jax.experimental.pallas import tpu_sc as plsc

N = 10000
NE = 320000
D = 128
DE = 16
PK = 32

NC = 2
NS = 16
NW = NC * NS
CH = 80
CPT = NE // (CH * NW)
NPAIR = CPT // 2
WB = 624
TAIL = N - NS * WB
TAIL0 = NS * WB

_DN = lax.GatherDimensionNumbers(
    offset_dims=(), collapsed_slice_dims=(0,), start_index_map=(0,))


def _lane_bcast(wv, j):
    return lax.gather(wv, jnp.full((16, 1), j, jnp.int32), _DN, (1,),
                      mode=lax.GatherScatterMode.PROMISE_IN_BOUNDS)


def _pool_body(v_ref, aw_ref, ab_ref, out_ref):
    x = v_ref[...]
    x = jnp.where(x >= 0.0, x, 0.2 * x)
    p = lax.dot_general(x, aw_ref[...], (((1,), (1,)), ((), ())),
                        preferred_element_type=jnp.float32)
    p = jnp.maximum(p + ab_ref[...], 0.0)
    lo = lax.bitcast_convert_type(p[:, :D // 2].astype(jnp.bfloat16),
                                  jnp.uint16).astype(jnp.uint32)
    hi = lax.bitcast_convert_type(p[:, D // 2:].astype(jnp.bfloat16),
                                  jnp.uint16).astype(jnp.uint32)
    out_ref[...] = lax.bitcast_convert_type(lo | (hi << 16), jnp.int32)


def _pool(V, A_w, A_b):
    return pl.pallas_call(
        _pool_body,
        out_shape=jax.ShapeDtypeStruct((N, D // 2), jnp.int32),
    )(V, A_w, A_b.reshape(1, D))


def _zero_init(z_hbm, acc_sh, sid):
    row0 = sid * WB
    pltpu.sync_copy(z_hbm.at[pl.ds(row0, WB)], acc_sh.at[pl.ds(row0, WB)])

    @pl.when(sid == NS - 1)
    def _zt():
        pltpu.sync_copy(z_hbm.at[pl.ds(TAIL0, TAIL)],
                        acc_sh.at[pl.ds(TAIL0, TAIL)])


def _writeback(acc_sh, out_hbm, cid, sid):
    row0 = sid * WB
    pltpu.sync_copy(acc_sh.at[pl.ds(row0, WB)],
                    out_hbm.at[cid, pl.ds(row0, WB)])

    @pl.when(sid == NS - 1)
    def _wt():
        pltpu.sync_copy(acc_sh.at[pl.ds(TAIL0, TAIL)],
                        out_hbm.at[cid, pl.ds(TAIL0, TAIL)])


def _s1_body(pooled_hbm, src_hbm, dst_hbm, w_hbm, z1_hbm, s1_out,
             s1_sh, srcv0, srcv1, dstv0, dstv1, wvm0, wvm1, rbf0, rbf1,
             r32_0, r32_1, sa0, sa1, sb0, sb1, sd0, sd1):
    cid = lax.axis_index("c")
    sid = lax.axis_index("s")
    wid = cid * NS + sid
    base0 = wid * CPT * CH

    _zero_init(z1_hbm, s1_sh, sid)
    plsc.subcore_barrier()

    def issue_a(slot, ci):
        srcv, sem = (srcv0, sa0) if slot == 0 else (srcv1, sa1)
        b = base0 + ci * CH
        pltpu.async_copy(src_hbm.at[pl.ds(b, CH)], srcv, sem)

    def wait_a(slot):
        srcv, sem = (srcv0, sa0) if slot == 0 else (srcv1, sa1)
        pltpu.make_async_copy(src_hbm.at[pl.ds(0, CH)], srcv, sem).wait()

    def issue_b(slot, ci):
        srcv, dstv, wvm, rbf, sem = (
            (srcv0, dstv0, wvm0, rbf0, sb0) if slot == 0
            else (srcv1, dstv1, wvm1, rbf1, sb1))
        b = base0 + ci * CH
        pltpu.async_copy(pooled_hbm.at[srcv], rbf, sem)
        pltpu.async_copy(dst_hbm.at[pl.ds(b, CH)], dstv, sem)
        pltpu.async_copy(w_hbm.at[pl.ds(b, CH)], wvm, sem)

    def wait_b(slot):
        srcv, dstv, wvm, rbf, sem = (
            (srcv0, dstv0, wvm0, rbf0, sb0) if slot == 0
            else (srcv1, dstv1, wvm1, rbf1, sb1))
        pltpu.make_async_copy(pooled_hbm.at[srcv], rbf, sem).wait()
        pltpu.make_async_copy(dst_hbm.at[pl.ds(0, CH)], dstv, sem).wait()
        pltpu.make_async_copy(w_hbm.at[pl.ds(0, CH)], wvm, sem).wait()

    def compute(slot):
        wvm, rbf, r32 = ((wvm0, rbf0, r32_0) if slot == 0
                         else (wvm1, rbf1, r32_1))

        def _group(g, _):
            wv = wvm[pl.ds(g * 16, 16)]
            for h in range(4):
                ws = [_lane_bcast(wv, h * 4 + t) for t in range(4)]
                es = [g * 16 + h * 4 + t for t in range(4)]
                pks = [[rbf[es[t], pl.ds(m * 16, 16)]
                        for m in range(D // 32)] for t in range(4)]
                himask = jnp.full((16,), -65536, jnp.int32)
                for t in range(4):
                    for m in range(D // 32):
                        x = pks[t][m]
                        a = plsc.bitcast(x << 16, jnp.float32)
                        bb = plsc.bitcast(x & himask, jnp.float32)
                        r32[es[t], pl.ds(m * 16, 16)] = a * ws[t]
                        r32[es[t], pl.ds(D // 2 + m * 16, 16)] = bb * ws[t]
            return 0

        lax.fori_loop(0, CH // 16, _group, 0)

    def issue_d(slot):
        dstv, r32, sem = (dstv0, r32_0, sd0) if slot == 0 else (dstv1,
                                                                r32_1, sd1)
        pltpu.async_copy(r32, s1_sh.at[dstv], sem, add=True)

    def wait_d(slot):
        dstv, r32, sem = (dstv0, r32_0, sd0) if slot == 0 else (dstv1,
                                                                r32_1, sd1)
        pltpu.make_async_copy(r32, s1_sh.at[dstv], sem).wait()

    issue_a(0, 0)
    issue_a(1, 1)
    wait_a(0)
    issue_b(0, 0)

    def _pair(p, _):
        wait_b(0)
        wait_a(1)
        issue_b(1, 2 * p + 1)
        issue_a(0, 2 * p + 2)

        @pl.when(p > 0)
        def _():
            wait_d(0)

        compute(0)
        issue_d(0)
        wait_b(1)
        wait_a(0)
        issue_b(0, 2 * p + 2)

        @pl.when(p < NPAIR - 1)
        def _():
            issue_a(1, 2 * p + 3)

        @pl.when(p > 0)
        def _():
            wait_d(1)

        compute(1)
        issue_d(1)
        return 0

    lax.fori_loop(0, NPAIR, _pair, 0)

    wait_b(0)
    wait_d(0)
    compute(0)
    issue_d(0)
    wait_d(0)
    wait_d(1)

    plsc.subcore_barrier()
    _writeback(s1_sh, s1_out, cid, sid)


def _sc_s1(pooled, src, dst, w):
    mesh = plsc.VectorSubcoreMesh(core_axis_name="c", subcore_axis_name="s")
    f = pl.kernel(
        _s1_body,
        out_type=jax.ShapeDtypeStruct((NC, N, D), jnp.float32),
        mesh=mesh,
        compiler_params=pltpu.CompilerParams(use_tc_tiling_on_sc=False,
                                             needs_layout_passes=False),
        scratch_types=[
            pltpu.VMEM_SHARED((N, D), jnp.float32),
            pltpu.VMEM((CH,), jnp.int32),
            pltpu.VMEM((CH,), jnp.int32),
            pltpu.VMEM((CH,), jnp.int32),
            pltpu.VMEM((CH,), jnp.int32),
            pltpu.VMEM((CH,), jnp.float32),
            pltpu.VMEM((CH,), jnp.float32),
            pltpu.VMEM((CH, D // 2), jnp.int32),
            pltpu.VMEM((CH, D // 2), jnp.int32),
            pltpu.VMEM((CH, D), jnp.float32),
            pltpu.VMEM((CH, D), jnp.float32),
            pltpu.SemaphoreType.DMA,
            pltpu.SemaphoreType.DMA,
            pltpu.SemaphoreType.DMA,
            pltpu.SemaphoreType.DMA,
            pltpu.SemaphoreType.DMA,
            pltpu.SemaphoreType.DMA,
        ],
    )
    z1 = jnp.zeros((N, D), jnp.float32)
    return f(pooled, src, dst, w, z1)


CH2 = 400
CPT2 = NE // (CH2 * NW)
NPAIR2 = CPT2 // 2
SPLITS = ((0, 96), (96, 96), (192, 96), (288, 112))


def _s2x_body(dst_hbm, w_hbm, e_hbm, z2_hbm, s2x_out,
              s2x_sh, da0, db0, dc0, dd0, da1, db1, dc1, dd1,
              wvm0, wvm1, ev0, ev1, pk0, pk1,
              sa0, sa1, sb0, sb1, sd0, sd1):
    cid = lax.axis_index("c")
    sid = lax.axis_index("s")
    wid = cid * NS + sid
    base0 = wid * CPT2 * CH2
    dsts0 = (da0, db0, dc0, dd0)
    dsts1 = (da1, db1, dc1, dd1)

    _zero_init(z2_hbm, s2x_sh, sid)
    plsc.subcore_barrier()

    lane = lax.iota(jnp.int32, 16)

    def issue_a(slot, ci):
        ev, wvm, sem = (ev0, wvm0, sa0) if slot == 0 else (ev1, wvm1, sa1)
        b = base0 + ci * CH2
        pltpu.async_copy(e_hbm.at[pl.ds(b * DE, CH2 * DE)], ev, sem)
        pltpu.async_copy(w_hbm.at[pl.ds(b, CH2)], wvm, sem)

    def wait_a(slot):
        ev, wvm, sem = (ev0, wvm0, sa0) if slot == 0 else (ev1, wvm1, sa1)
        pltpu.make_async_copy(e_hbm.at[pl.ds(0, CH2 * DE)], ev, sem).wait()
        pltpu.make_async_copy(w_hbm.at[pl.ds(0, CH2)], wvm, sem).wait()

    def issue_b(slot, ci):
        dsts, sem = (dsts0, sb0) if slot == 0 else (dsts1, sb1)
        b = base0 + ci * CH2
        for (off, ln), dv in zip(SPLITS, dsts):
            pltpu.async_copy(dst_hbm.at[pl.ds(b + off, ln)], dv, sem)

    def wait_b(slot):
        dsts, sem = (dsts0, sb0) if slot == 0 else (dsts1, sb1)
        for (off, ln), dv in zip(SPLITS, dsts):
            pltpu.make_async_copy(dst_hbm.at[pl.ds(0, ln)], dv, sem).wait()

    def compute(slot):
        wvm, ev, pk = (wvm0, ev0, pk0) if slot == 0 else (wvm1, ev1, pk1)

        def _group(g, _):
            wv = wvm[pl.ds(g * 16, 16)]
            for h in range(4):
                ws = [_lane_bcast(wv, h * 4 + t) for t in range(4)]
                es = [g * 16 + h * 4 + t for t in range(4)]
                evs = [ev[pl.ds(es[t] * DE, DE)] for t in range(4)]
                for t in range(4):
                    pk[es[t], pl.ds(0, 16)] = evs[t] * ws[t]
                    pk[es[t], pl.ds(16, 16)] = jnp.where(
                        lane == 0, ws[t], jnp.where(lane == 1, 1.0, 0.0))
            return 0

        lax.fori_loop(0, CH2 // 16, _group, 0)

    def issue_d(slot):
        dsts, pk, sem = ((dsts0, pk0, sd0) if slot == 0
                         else (dsts1, pk1, sd1))
        for (off, ln), dv in zip(SPLITS, dsts):
            pltpu.async_copy(pk.at[pl.ds(off, ln)], s2x_sh.at[dv], sem,
                             add=True)

    def wait_d(slot):
        dsts, pk, sem = ((dsts0, pk0, sd0) if slot == 0
                         else (dsts1, pk1, sd1))
        for (off, ln), dv in zip(SPLITS, dsts):
            pltpu.make_async_copy(pk.at[pl.ds(off, ln)], s2x_sh.at[dv],
                                  sem).wait()

    issue_a(0, 0)
    issue_a(1, 1)
    wait_a(0)
    issue_b(0, 0)

    def _pair(p, _):
        wait_b(0)
        compute(0)
        issue_d(0)
        wait_a(1)

        @pl.when(p > 0)
        def _():
            wait_d(1)

        issue_b(1, 2 * p + 1)
        issue_a(0, 2 * p + 2)
        wait_b(1)
        compute(1)
        issue_d(1)
        wait_a(0)
        wait_d(0)
        issue_b(0, 2 * p + 2)

        @pl.when(p < NPAIR2 - 1)
        def _():
            issue_a(1, 2 * p + 3)

        return 0

    lax.fori_loop(0, NPAIR2, _pair, 0)

    wait_b(0)
    compute(0)
    issue_d(0)
    wait_d(0)
    wait_d(1)

    plsc.subcore_barrier()
    _writeback(s2x_sh, s2x_out, cid, sid)


def _sc_s2x(dst, w, E):
    mesh = plsc.VectorSubcoreMesh(core_axis_name="c", subcore_axis_name="s")
    f = pl.kernel(
        _s2x_body,
        out_type=jax.ShapeDtypeStruct((NC, N, PK), jnp.float32),
        mesh=mesh,
        compiler_params=pltpu.CompilerParams(use_tc_tiling_on_sc=False),
        scratch_types=(
            [pltpu.VMEM_SHARED((N, PK), jnp.float32)]
            + [pltpu.VMEM((ln,), jnp.int32)
               for _s in range(2) for (_o, ln) in SPLITS]
            + [
                pltpu.VMEM((CH2,), jnp.float32),
                pltpu.VMEM((CH2,), jnp.float32),
                pltpu.VMEM((CH2 * DE,), jnp.float32),
                pltpu.VMEM((CH2 * DE,), jnp.float32),
                pltpu.VMEM((CH2, PK), jnp.float32),
                pltpu.VMEM((CH2, PK), jnp.float32),
            ]
            + [pltpu.SemaphoreType.DMA] * 6
        ),
    )
    z2 = jnp.zeros((N, PK), jnp.float32)
    return f(dst, w, E.reshape(NE * DE), z2)


def _final_body(s1_ref, s2x_ref, vin_ref, mw_ref, mb_ref, w1_ref, w2_ref,
                wb_ref, out_ref):
    s1 = s1_ref[0] + s1_ref[1]
    s2x = s2x_ref[0] + s2x_ref[1]
    s2 = s2x[:, :DE]
    s3 = s2x[:, DE:DE + 1]
    cnt = s2x[:, DE + 1:DE + 2]
    proj = lax.dot_general(s2, mw_ref[...], (((1,), (1,)), ((), ())),
                           preferred_element_type=jnp.float32)
    summed = s1 + proj + s3 * mb_ref[...]
    agg = summed / jnp.maximum(cnt, 1.0)
    o = lax.dot_general(agg, w1_ref[...], (((1,), (1,)), ((), ())),
                        preferred_element_type=jnp.float32)
    o = o + lax.dot_general(vin_ref[...], w2_ref[...],
                            (((1,), (1,)), ((), ())),
                            preferred_element_type=jnp.float32)
    out_ref[...] = jnp.maximum(o + wb_ref[...], 0.0)


def _final(S1p, S2xp, V_in, M_w, M_b, W1, W2, W_b):
    return pl.pallas_call(
        _final_body,
        out_shape=jax.ShapeDtypeStruct((N, D), jnp.float32),
    )(S1p, S2xp, V_in, M_w, M_b.reshape(1, D), W1, W2, W_b.reshape(1, D))


def kernel(V, V_in, E, edge_attr, edge_index, A_w, A_b, M_w, M_b, W_w, W_b):
    w = edge_attr[:, 0]
    src = edge_index[0]
    dst = edge_index[1]
    pooled = _pool(V, A_w, A_b)
    S1p = _sc_s1(pooled, src, dst, w)
    S2xp = _sc_s2x(dst, w, E)
    return _final(S1p, S2xp, V_in, M_w, M_b, W_w[:, :D], W_w[:, D:], W_b)
